# baseline (speedup 1.0000x reference)
import jax
import jax.numpy as jnp
from jax import lax
from jax.experimental import pallas as pl
from jax.experimental.pallas import tpu as pltpu
from jax.experimental.pallas import tpu_sc as plsc

N = 10000
E = 320000
F_IN = 128
H_DIM = 64

NC = 2
NS = 16
NW = NC * NS
NP = 10240
TS = NP // NS

E_PAD = 327680
E_ALL = E_PAD + 2048
EPT = E_PAD // NS
CH = 128
SUP = 1024
NSUP = EPT // SUP
CHA = 1024
NCHA = (E_PAD // NC // NS) // CHA

_f32 = jnp.float32
_i32 = jnp.int32


def _deg_body(col2d, ew_hbm, deg0_out, deg1_out,
              deg_sh, colA, ewA, zd):
    c = lax.axis_index("c")
    s = lax.axis_index("s")

    def zerod(i, carry):
        zd[pl.ds(i * 16, 16)] = jnp.zeros((16,), _f32)
        return carry
    lax.fori_loop(0, TS // 16, zerod, None)
    pltpu.sync_copy(zd, deg_sh.at[pl.ds(s * TS, TS)])
    plsc.subcore_barrier()

    def chunk(j, carry):
        base = (c * NS + s) * (E_PAD // NW) + j * CHA
        pltpu.sync_copy(
            col2d.at[pl.ds(pl.multiple_of(base // 128, 8), CHA // 128)], colA)
        pltpu.sync_copy(ew_hbm.at[pl.ds(pl.multiple_of(base, 8), CHA)], ewA)
        for jj in range(CHA // 128):
            pltpu.sync_copy(ewA.at[pl.ds(jj * 128, 128)],
                            deg_sh.at[colA.at[jj]], add=True)
        return carry
    lax.fori_loop(0, NCHA, chunk, None)
    plsc.subcore_barrier()

    @pl.when(c == 0)
    def _():
        pltpu.sync_copy(deg_sh.at[pl.ds(s * TS, TS)],
                        deg0_out.at[pl.ds(s * TS, TS)])

    @pl.when(c == 1)
    def _():
        pltpu.sync_copy(deg_sh.at[pl.ds(s * TS, TS)],
                        deg1_out.at[pl.ds(s * TS, TS)])


def _sc_degrees(col2d, ew_pad):
    kern = pl.kernel(
        _deg_body,
        out_type=[
            jax.ShapeDtypeStruct((NP,), _f32),
            jax.ShapeDtypeStruct((NP,), _f32),
        ],
        mesh=plsc.VectorSubcoreMesh(core_axis_name="c", subcore_axis_name="s"),
        compiler_params=pltpu.CompilerParams(needs_layout_passes=False),
        scratch_types=[
            pltpu.VMEM_SHARED((NP,), _f32),
            pltpu.VMEM((CHA // 128, 128), _i32),
            pltpu.VMEM((CHA,), _f32),
            pltpu.VMEM((TS,), _f32),
        ],
    )
    return kern(col2d, ew_pad)


def _prescale_body(deg0, deg1, x, xp2_ref, dinv_ref):
    dv = lax.rsqrt(deg0[...] + deg1[...] + 1.0)
    dinv_ref[...] = dv
    xp2_ref[0] = dv * x[:, :H_DIM]
    xp2_ref[1] = dv * x[:, H_DIM:]


def _tc_prescale(deg0c, deg1c, x):
    BR = 2000
    row = lambda i: (i, 0)
    return pl.pallas_call(
        _prescale_body,
        grid=(N // BR,),
        in_specs=[
            pl.BlockSpec((BR, 1), row),
            pl.BlockSpec((BR, 1), row),
            pl.BlockSpec((BR, F_IN), row),
        ],
        out_specs=[
            pl.BlockSpec((2, BR, H_DIM), lambda i: (0, i, 0)),
            pl.BlockSpec((BR, 1), row),
        ],
        out_shape=[
            jax.ShapeDtypeStruct((2, N, H_DIM), _f32),
            jax.ShapeDtypeStruct((N, 1), _f32),
        ],
    )(deg0c, deg1c, x)


def _prop_body(row2d, col2d, ew_hbm, xp2_hbm,
               s_out,
               s_sh, za,
               rowi0, rowi1, coli0, coli1, ewb0, ewb1,
               rows0, rows1, rows2, rows3,
               sg0, sg1, sg2, sg3, ss0, ss1, ss2, ss3, semi0, semi1):
    c = lax.axis_index("c")
    s = lax.axis_index("s")
    rowi_b = (rowi0, rowi1)
    coli_b = (coli0, coli1)
    ew_b = (ewb0, ewb1)
    rows_b = (rows0, rows1, rows2, rows3)
    semg_b = (sg0, sg1, sg2, sg3)
    sems_b = (ss0, ss1, ss2, ss3)
    semi_b = (semi0, semi1)
    coff = c * N

    def zeroa(i, carry):
        for jj in range(H_DIM // 16):
            za[i, pl.ds(jj * 16, 16)] = jnp.zeros((16,), _f32)
        return carry
    lax.fori_loop(0, 16, zeroa, None)
    for kk in range(TS // 16):
        pltpu.sync_copy(za, s_sh.at[pl.ds(s * TS + kk * 16, 16)])
    plsc.subcore_barrier()

    def stage_start(sc, b):
        base = pl.multiple_of(s * (EPT // 128) + sc * (SUP // 128), 8)
        pltpu.async_copy(row2d.at[pl.ds(base, SUP // 128)], rowi_b[b],
                         semi_b[b])
        pltpu.async_copy(col2d.at[pl.ds(base, SUP // 128)], coli_b[b],
                         semi_b[b])
        ebase = pl.multiple_of(s * EPT + sc * SUP, 8)
        pltpu.async_copy(ew_hbm.at[pl.ds(ebase, SUP)], ew_b[b], semi_b[b])

    def stage_wait(sc, b):
        base = pl.multiple_of(s * (EPT // 128) + sc * (SUP // 128), 8)
        pltpu.make_async_copy(row2d.at[pl.ds(base, SUP // 128)], rowi_b[b],
                              semi_b[b]).wait()
        pltpu.make_async_copy(col2d.at[pl.ds(base, SUP // 128)], coli_b[b],
                              semi_b[b]).wait()
        ebase = pl.multiple_of(s * EPT + sc * SUP, 8)
        pltpu.make_async_copy(ew_hbm.at[pl.ds(ebase, SUP)], ew_b[b],
                              semi_b[b]).wait()
        def adj(t, carry):
            for v in range(8):
                rowi_b[b][t, pl.ds(v * 16, 16)] = (
                    rowi_b[b][t, pl.ds(v * 16, 16)] + coff)
            return carry
        lax.fori_loop(0, SUP // 128, adj, None)

    def gather_start(ib, sub, gb):
        pltpu.async_copy(xp2_hbm.at[rowi_b[ib].at[sub]], rows_b[gb],
                         semg_b[gb])

    def gather_wait(ib, sub, gb):
        pltpu.make_async_copy(xp2_hbm.at[rowi_b[ib].at[sub]], rows_b[gb],
                              semg_b[gb]).wait()

    def scatter_start(ib, sub, gb):
        pltpu.async_copy(rows_b[gb], s_sh.at[coli_b[ib].at[sub]],
                         sems_b[gb], add=True)

    def scatter_wait(ib, sub, gb):
        pltpu.make_async_copy(rows_b[gb], s_sh.at[coli_b[ib].at[sub]],
                              sems_b[gb]).wait()

    stage_start(jnp.int32(0), 0)
    stage_wait(jnp.int32(0), 0)
    gather_start(0, 0, 0)
    gather_start(0, 1, 1)
    stage_start(jnp.int32(1), 1)

    NSUB = SUP // CH

    def super_body(so, carry):
      for ib in range(2):
        sc = so * 2 + ib
        for sub in range(NSUB):
            gb = sub % 4
            gather_wait(ib, sub, gb)

            rows_ref = rows_b[gb]

            def mul_body(k0, inner):
                f16 = ew_b[ib][pl.ds(sub * CH + k0 * 16, 16)]
                for l in range(16):
                    k = k0 * 16 + l
                    fs = f16[l]
                    for jj in range(H_DIM // 16):
                        rows_ref[k, pl.ds(jj * 16, 16)] = (
                            rows_ref[k, pl.ds(jj * 16, 16)] * fs)
                return inner
            lax.fori_loop(0, CH // 16, mul_body, None)

            ngb = (sub + 2) % 4
            if sub < 2:
                prev_ib = 1 - ib
                prev_sub = NSUB - 2 + sub

                @pl.when(sc > 0)
                def _():
                    scatter_wait(prev_ib, prev_sub, ngb)
            else:
                scatter_wait(ib, sub - 2, ngb)
            if sub < NSUB - 2:
                gather_start(ib, sub + 2, ngb)
            else:
                nsub = sub + 2 - NSUB
                if nsub == 0:
                    stage_wait(sc + 1, 1 - ib)
                gather_start(1 - ib, nsub, ngb)

            scatter_start(ib, sub, gb)
        stage_start(sc + 2, ib)
      return carry
    lax.fori_loop(0, NSUP // 2, super_body, None)

    ibf = NSUP % 2
    gather_wait(ibf, 0, 0)
    gather_wait(ibf, 1, 1)
    scatter_wait(1 - ibf, NSUB - 2, 2)
    scatter_wait(1 - ibf, NSUB - 1, 3)
    stage_wait(NSUP + 1, 1 - ibf)
    plsc.subcore_barrier()

    pltpu.sync_copy(
        s_sh.at[pl.ds(s * TS, TS)],
        s_out.at[pl.ds(pl.multiple_of(c * NP + s * TS, 8), TS)])


def _sc_propagate(row2d, col2d, ew_pad, xp2):
    kern = pl.kernel(
        _prop_body,
        out_type=[
            jax.ShapeDtypeStruct((2 * NP, H_DIM), _f32),
        ],
        mesh=plsc.VectorSubcoreMesh(core_axis_name="c", subcore_axis_name="s"),
        compiler_params=pltpu.CompilerParams(needs_layout_passes=False,
                                             use_tc_tiling_on_sc=False),
        scratch_types=[
            pltpu.VMEM_SHARED((NP, H_DIM), _f32),
            pltpu.VMEM((16, H_DIM), _f32),
            pltpu.VMEM((SUP // 128, 128), _i32),
            pltpu.VMEM((SUP // 128, 128), _i32),
            pltpu.VMEM((SUP // 128, 128), _i32),
            pltpu.VMEM((SUP // 128, 128), _i32),
            pltpu.VMEM((SUP,), _f32),
            pltpu.VMEM((SUP,), _f32),
            pltpu.VMEM((CH, H_DIM), _f32),
            pltpu.VMEM((CH, H_DIM), _f32),
            pltpu.VMEM((CH, H_DIM), _f32),
            pltpu.VMEM((CH, H_DIM), _f32),
            pltpu.SemaphoreType.DMA,
            pltpu.SemaphoreType.DMA,
            pltpu.SemaphoreType.DMA,
            pltpu.SemaphoreType.DMA,
            pltpu.SemaphoreType.DMA,
            pltpu.SemaphoreType.DMA,
            pltpu.SemaphoreType.DMA,
            pltpu.SemaphoreType.DMA,
            pltpu.SemaphoreType.DMA,
            pltpu.SemaphoreType.DMA,
        ],
    )
    return kern(row2d, col2d, ew_pad, xp2)


def _tc_body(sl, sr, dinv, xpl, xpr, h,
             wc, bc, wbd, whz, bzr, wh_b, bh,
             w_out, b_out, y_ref, hn_ref):
    dv = dinv[...]
    xa = dv * jnp.concatenate(
        [sl[...] + xpl[...], sr[...] + xpr[...]], axis=1)
    hh = h[...]
    c = jnp.dot(xa, wc[...]) + bc[...]
    g = jnp.dot(c, wbd[...])
    t = jnp.dot(hh, whz[...]) + bzr[...]
    z = jax.nn.sigmoid(g[:, :H_DIM] + t[:, :H_DIM])
    r = jax.nn.sigmoid(g[:, H_DIM:2 * H_DIM] + t[:, H_DIM:])
    ht = jnp.tanh(g[:, 2 * H_DIM:] + jnp.dot(hh * r, wh_b[...]) + bh[...])
    hn = z * hh + (1.0 - z) * ht
    hn_ref[...] = hn
    y_ref[...] = jnp.dot(jax.nn.relu(hn), w_out[...]) + b_out[...]


def _tc_dense(sl, sr, dinv2d, xpl, xpr, h, wc, bc, wbd, whz, bzr, wh_b, bh,
              w_out, b_out):
    BR = 2000
    row = lambda i: (i, 0)
    rep = lambda i: (0, 0)
    return pl.pallas_call(
        _tc_body,
        grid=(N // BR,),
        in_specs=[
            pl.BlockSpec((BR, H_DIM), row),
            pl.BlockSpec((BR, H_DIM), row),
            pl.BlockSpec((BR, 1), row),
            pl.BlockSpec((BR, H_DIM), row),
            pl.BlockSpec((BR, H_DIM), row),
            pl.BlockSpec((BR, H_DIM), row),
            pl.BlockSpec((F_IN, 3 * H_DIM), rep),
            pl.BlockSpec((1, 3 * H_DIM), rep),
            pl.BlockSpec((3 * H_DIM, 3 * H_DIM), rep),
            pl.BlockSpec((H_DIM, 2 * H_DIM), rep),
            pl.BlockSpec((1, 2 * H_DIM), rep),
            pl.BlockSpec((H_DIM, H_DIM), rep),
            pl.BlockSpec((1, H_DIM), rep),
            pl.BlockSpec((H_DIM, F_IN), rep),
            pl.BlockSpec((1, F_IN), rep),
        ],
        out_specs=[
            pl.BlockSpec((BR, F_IN), row),
            pl.BlockSpec((BR, H_DIM), row),
        ],
        out_shape=[
            jax.ShapeDtypeStruct((N, F_IN), _f32),
            jax.ShapeDtypeStruct((N, H_DIM), _f32),
        ],
    )(sl, sr, dinv2d, xpl, xpr, h, wc, bc, wbd, whz, bzr, wh_b, bh,
      w_out, b_out)


def kernel(g_edge_index, node_feat, edge_weight, hidden_state,
           W_cz, b_cz, Wz, bz, W_cr, b_cr, Wr, br,
           W_ch, b_ch, Wh, bh, W_out, b_out):
    row = g_edge_index[0]
    col = g_edge_index[1]

    npad = E_ALL - E
    pad_idx = (lax.iota(_i32, npad) * 37) % N
    row_p = jnp.concatenate([row, pad_idx])
    col_p = jnp.concatenate([col, pad_idx])
    ew_p = jnp.concatenate([edge_weight, jnp.zeros((npad,), _f32)])
    row2d = row_p.reshape(E_ALL // 128, 128)
    col2d = col_p.reshape(E_ALL // 128, 128)

    deg0, deg1 = _sc_degrees(col2d, ew_p)
    xp2, dinv2d = _tc_prescale(deg0[:N].reshape(N, 1),
                               deg1[:N].reshape(N, 1), node_feat)
    s_both = _sc_propagate(row2d, col2d, ew_p, xp2.reshape(2 * N, H_DIM))[0]

    zed = jnp.zeros((H_DIM, H_DIM), _f32)
    wc = jnp.concatenate([W_cz, W_cr, W_ch], axis=1)
    bc = jnp.concatenate([b_cz, b_cr, b_ch]).reshape(1, 3 * H_DIM)
    wbd = jnp.concatenate([
        jnp.concatenate([Wz[:H_DIM], zed, zed], axis=1),
        jnp.concatenate([zed, Wr[:H_DIM], zed], axis=1),
        jnp.concatenate([zed, zed, Wh[:H_DIM]], axis=1)], axis=0)
    whz = jnp.concatenate([Wz[H_DIM:], Wr[H_DIM:]], axis=1)
    bzr = jnp.concatenate([bz, br]).reshape(1, 2 * H_DIM)
    y, hn = _tc_dense(
        s_both[:N], s_both[NP:NP + N], dinv2d, xp2[0], xp2[1], hidden_state,
        wc, bc, wbd, whz, bzr, Wh[H_DIM:], bh.reshape(1, H_DIM),
        W_out, b_out.reshape(1, F_IN))
    return (y, hn)


# speedup vs baseline: 1.0312x; 1.0312x over previous
import jax
import jax.numpy as jnp
from jax import lax
from jax.experimental import pallas as pl
from jax.experimental.pallas import tpu as pltpu
from jax.experimental.pallas import tpu_sc as plsc

N = 10000
E = 320000
F_IN = 128
H_DIM = 64

NC = 2
NS = 16
NW = NC * NS
NP = 10240
TS = NP // NS

E_PAD = 327680
E_ALL = E_PAD + 2048
EPW = E_PAD // NW
CH = 64
SUP = 1024
NSUP = EPW // SUP
CHA = 1024
NCHA = (E_PAD // NC // NS) // CHA

_f32 = jnp.float32
_i32 = jnp.int32


def _deg_body(col2d, ew_hbm, deg0_out, deg1_out,
              deg_sh, colA, ewA, zd):
    c = lax.axis_index("c")
    s = lax.axis_index("s")

    def zerod(i, carry):
        zd[pl.ds(i * 16, 16)] = jnp.zeros((16,), _f32)
        return carry
    lax.fori_loop(0, TS // 16, zerod, None)
    pltpu.sync_copy(zd, deg_sh.at[pl.ds(s * TS, TS)])
    plsc.subcore_barrier()

    def chunk(j, carry):
        base = (c * NS + s) * (E_PAD // NW) + j * CHA
        pltpu.sync_copy(
            col2d.at[pl.ds(pl.multiple_of(base // 128, 8), CHA // 128)], colA)
        pltpu.sync_copy(ew_hbm.at[pl.ds(pl.multiple_of(base, 8), CHA)], ewA)
        for jj in range(CHA // 128):
            pltpu.sync_copy(ewA.at[pl.ds(jj * 128, 128)],
                            deg_sh.at[colA.at[jj]], add=True)
        return carry
    lax.fori_loop(0, NCHA, chunk, None)
    plsc.subcore_barrier()

    @pl.when(c == 0)
    def _():
        pltpu.sync_copy(deg_sh.at[pl.ds(s * TS, TS)],
                        deg0_out.at[pl.ds(s * TS, TS)])

    @pl.when(c == 1)
    def _():
        pltpu.sync_copy(deg_sh.at[pl.ds(s * TS, TS)],
                        deg1_out.at[pl.ds(s * TS, TS)])


def _sc_degrees(col2d, ew_pad):
    kern = pl.kernel(
        _deg_body,
        out_type=[
            jax.ShapeDtypeStruct((NP,), _f32),
            jax.ShapeDtypeStruct((NP,), _f32),
        ],
        mesh=plsc.VectorSubcoreMesh(core_axis_name="c", subcore_axis_name="s"),
        compiler_params=pltpu.CompilerParams(needs_layout_passes=False),
        scratch_types=[
            pltpu.VMEM_SHARED((NP,), _f32),
            pltpu.VMEM((CHA // 128, 128), _i32),
            pltpu.VMEM((CHA,), _f32),
            pltpu.VMEM((TS,), _f32),
        ],
    )
    return kern(col2d, ew_pad)


def _prescale_body(deg0, deg1, x, xp_ref, dinv_ref):
    dv = lax.rsqrt(deg0[...] + deg1[...] + 1.0)
    dinv_ref[...] = dv
    xp_ref[...] = dv * x[...]


def _tc_prescale(deg0c, deg1c, x):
    BR = 2000
    row = lambda i: (i, 0)
    return pl.pallas_call(
        _prescale_body,
        grid=(N // BR,),
        in_specs=[
            pl.BlockSpec((BR, 1), row),
            pl.BlockSpec((BR, 1), row),
            pl.BlockSpec((BR, F_IN), row),
        ],
        out_specs=[
            pl.BlockSpec((BR, F_IN), row),
            pl.BlockSpec((BR, 1), row),
        ],
        out_shape=[
            jax.ShapeDtypeStruct((N, F_IN), _f32),
            jax.ShapeDtypeStruct((N, 1), _f32),
        ],
    )(deg0c, deg1c, x)


def _prop_body(row2d, col2d, ew_hbm, xp_hbm,
               s_out,
               s_sh, za,
               rowi0, rowi1, coli0, coli1, ewb0, ewb1,
               rows0, rows1, rows2, rows3,
               sg0, sg1, sg2, sg3, ss0, ss1, ss2, ss3, semi0, semi1):
    c = lax.axis_index("c")
    s = lax.axis_index("s")
    wid = s * NC + c
    rowi_b = (rowi0, rowi1)
    coli_b = (coli0, coli1)
    ew_b = (ewb0, ewb1)
    rows_b = (rows0, rows1, rows2, rows3)
    semg_b = (sg0, sg1, sg2, sg3)
    sems_b = (ss0, ss1, ss2, ss3)
    semi_b = (semi0, semi1)

    def zeroa(i, carry):
        for jj in range(F_IN // 16):
            za[i, pl.ds(jj * 16, 16)] = jnp.zeros((16,), _f32)
        return carry
    lax.fori_loop(0, 8, zeroa, None)
    for kk in range(TS // 8):
        pltpu.sync_copy(za, s_sh.at[pl.ds(s * TS + kk * 8, 8)])
    plsc.subcore_barrier()

    def stage_start(sc, b):
        base = pl.multiple_of(wid * (EPW // 64) + sc * (SUP // 64), 8)
        pltpu.async_copy(row2d.at[pl.ds(base, SUP // 64)], rowi_b[b],
                         semi_b[b])
        pltpu.async_copy(col2d.at[pl.ds(base, SUP // 64)], coli_b[b],
                         semi_b[b])
        ebase = pl.multiple_of(wid * EPW + sc * SUP, 8)
        pltpu.async_copy(ew_hbm.at[pl.ds(ebase, SUP)],
                         ew_b[b].at[pl.ds(0, SUP)], semi_b[b])

    def stage_wait(sc, b):
        base = pl.multiple_of(wid * (EPW // 64) + sc * (SUP // 64), 8)
        pltpu.make_async_copy(row2d.at[pl.ds(base, SUP // 64)], rowi_b[b],
                              semi_b[b]).wait()
        pltpu.make_async_copy(col2d.at[pl.ds(base, SUP // 64)], coli_b[b],
                              semi_b[b]).wait()
        ebase = pl.multiple_of(wid * EPW + sc * SUP, 8)
        pltpu.make_async_copy(ew_hbm.at[pl.ds(ebase, SUP)],
                              ew_b[b].at[pl.ds(0, SUP)], semi_b[b]).wait()

    def gather_start(ib, sub, gb):
        pltpu.async_copy(xp_hbm.at[rowi_b[ib].at[sub]], rows_b[gb],
                         semg_b[gb])

    def gather_wait(ib, sub, gb):
        pltpu.make_async_copy(xp_hbm.at[rowi_b[ib].at[sub]], rows_b[gb],
                              semg_b[gb]).wait()

    def scatter_start(ib, sub, gb):
        pltpu.async_copy(rows_b[gb], s_sh.at[coli_b[ib].at[sub]],
                         sems_b[gb], add=True)

    def scatter_wait(ib, sub, gb):
        pltpu.make_async_copy(rows_b[gb], s_sh.at[coli_b[ib].at[sub]],
                              sems_b[gb]).wait()

    stage_start(jnp.int32(0), 0)
    stage_wait(jnp.int32(0), 0)
    gather_start(0, 0, 0)
    gather_start(0, 1, 1)
    stage_start(jnp.int32(1), 1)

    NSUB = SUP // CH

    def super_body(so, carry):
      for ib in range(2):
        sc = so * 2 + ib
        for sub in range(NSUB):
            gb = sub % 4
            gather_wait(ib, sub, gb)

            rows_ref = rows_b[gb]

            def mul_body(k0, inner):
                f16 = ew_b[ib][pl.ds(sub * CH + k0 * 8, 16)]
                for l in range(8):
                    k = k0 * 8 + l
                    fs = f16[l]
                    for jj in range(F_IN // 16):
                        rows_ref[k, pl.ds(jj * 16, 16)] = (
                            rows_ref[k, pl.ds(jj * 16, 16)] * fs)
                return inner
            lax.fori_loop(0, CH // 8, mul_body, None)

            ngb = (sub + 2) % 4
            if sub < 2:
                prev_ib = 1 - ib
                prev_sub = NSUB - 2 + sub

                @pl.when(sc > 0)
                def _():
                    scatter_wait(prev_ib, prev_sub, ngb)
            else:
                scatter_wait(ib, sub - 2, ngb)
            if sub < NSUB - 2:
                gather_start(ib, sub + 2, ngb)
            else:
                nsub = sub + 2 - NSUB
                if nsub == 0:
                    stage_wait(sc + 1, 1 - ib)
                gather_start(1 - ib, nsub, ngb)

            scatter_start(ib, sub, gb)
        stage_start(sc + 2, ib)
      return carry
    lax.fori_loop(0, NSUP // 2, super_body, None)

    ibf = NSUP % 2
    gather_wait(ibf, 0, 0)
    gather_wait(ibf, 1, 1)
    scatter_wait(1 - ibf, NSUB - 2, 2)
    scatter_wait(1 - ibf, NSUB - 1, 3)
    stage_wait(NSUP + 1, 1 - ibf)
    plsc.subcore_barrier()

    pltpu.sync_copy(
        s_sh.at[pl.ds(s * TS, TS)],
        s_out.at[pl.ds(pl.multiple_of(c * NP + s * TS, 8), TS)])


def _sc_propagate(row2d, col2d, ew_pad, xp):
    kern = pl.kernel(
        _prop_body,
        out_type=[
            jax.ShapeDtypeStruct((2 * NP, F_IN), _f32),
        ],
        mesh=plsc.VectorSubcoreMesh(core_axis_name="c", subcore_axis_name="s"),
        compiler_params=pltpu.CompilerParams(needs_layout_passes=False,
                                             use_tc_tiling_on_sc=False),
        scratch_types=[
            pltpu.VMEM_SHARED((NP, F_IN), _f32),
            pltpu.VMEM((8, F_IN), _f32),
            pltpu.VMEM((SUP // 64, 64), _i32),
            pltpu.VMEM((SUP // 64, 64), _i32),
            pltpu.VMEM((SUP // 64, 64), _i32),
            pltpu.VMEM((SUP // 64, 64), _i32),
            pltpu.VMEM((SUP + 16,), _f32),
            pltpu.VMEM((SUP + 16,), _f32),
            pltpu.VMEM((CH, F_IN), _f32),
            pltpu.VMEM((CH, F_IN), _f32),
            pltpu.VMEM((CH, F_IN), _f32),
            pltpu.VMEM((CH, F_IN), _f32),
            pltpu.SemaphoreType.DMA,
            pltpu.SemaphoreType.DMA,
            pltpu.SemaphoreType.DMA,
            pltpu.SemaphoreType.DMA,
            pltpu.SemaphoreType.DMA,
            pltpu.SemaphoreType.DMA,
            pltpu.SemaphoreType.DMA,
            pltpu.SemaphoreType.DMA,
            pltpu.SemaphoreType.DMA,
            pltpu.SemaphoreType.DMA,
        ],
    )
    return kern(row2d, col2d, ew_pad, xp)


def _tc_body(s0, s1, dinv, xp, h,
             wc, bc, wbd, whz, bzr, wh_b, bh,
             w_out, b_out, y_ref, hn_ref):
    dv = dinv[...]
    xa = dv * (s0[...] + s1[...] + xp[...])
    hh = h[...]
    c = jnp.dot(xa, wc[...]) + bc[...]
    g = jnp.dot(c, wbd[...])
    t = jnp.dot(hh, whz[...]) + bzr[...]
    z = jax.nn.sigmoid(g[:, :H_DIM] + t[:, :H_DIM])
    r = jax.nn.sigmoid(g[:, H_DIM:2 * H_DIM] + t[:, H_DIM:])
    ht = jnp.tanh(g[:, 2 * H_DIM:] + jnp.dot(hh * r, wh_b[...]) + bh[...])
    hn = z * hh + (1.0 - z) * ht
    hn_ref[...] = hn
    y_ref[...] = jnp.dot(jax.nn.relu(hn), w_out[...]) + b_out[...]


def _tc_dense(s0, s1, dinv2d, xp, h, wc, bc, wbd, whz, bzr, wh_b, bh,
              w_out, b_out):
    BR = 2000
    row = lambda i: (i, 0)
    rep = lambda i: (0, 0)
    return pl.pallas_call(
        _tc_body,
        grid=(N // BR,),
        in_specs=[
            pl.BlockSpec((BR, F_IN), row),
            pl.BlockSpec((BR, F_IN), row),
            pl.BlockSpec((BR, 1), row),
            pl.BlockSpec((BR, F_IN), row),
            pl.BlockSpec((BR, H_DIM), row),
            pl.BlockSpec((F_IN, 3 * H_DIM), rep),
            pl.BlockSpec((1, 3 * H_DIM), rep),
            pl.BlockSpec((3 * H_DIM, 3 * H_DIM), rep),
            pl.BlockSpec((H_DIM, 2 * H_DIM), rep),
            pl.BlockSpec((1, 2 * H_DIM), rep),
            pl.BlockSpec((H_DIM, H_DIM), rep),
            pl.BlockSpec((1, H_DIM), rep),
            pl.BlockSpec((H_DIM, F_IN), rep),
            pl.BlockSpec((1, F_IN), rep),
        ],
        out_specs=[
            pl.BlockSpec((BR, F_IN), row),
            pl.BlockSpec((BR, H_DIM), row),
        ],
        out_shape=[
            jax.ShapeDtypeStruct((N, F_IN), _f32),
            jax.ShapeDtypeStruct((N, H_DIM), _f32),
        ],
    )(s0, s1, dinv2d, xp, h, wc, bc, wbd, whz, bzr, wh_b, bh, w_out, b_out)


def kernel(g_edge_index, node_feat, edge_weight, hidden_state,
           W_cz, b_cz, Wz, bz, W_cr, b_cr, Wr, br,
           W_ch, b_ch, Wh, bh, W_out, b_out):
    row = g_edge_index[0]
    col = g_edge_index[1]

    npad = E_ALL - E
    pad_idx = (lax.iota(_i32, npad) * 37) % N
    row_p = jnp.concatenate([row, pad_idx])
    col_p = jnp.concatenate([col, pad_idx])
    ew_p = jnp.concatenate([edge_weight, jnp.zeros((npad,), _f32)])
    col2d_t = col_p.reshape(E_ALL // 128, 128)
    row2d = row_p.reshape(E_ALL // 64, 64)
    col2d = col_p.reshape(E_ALL // 64, 64)

    deg0, deg1 = _sc_degrees(col2d_t, ew_p)
    xp, dinv2d = _tc_prescale(deg0[:N].reshape(N, 1),
                              deg1[:N].reshape(N, 1), node_feat)
    s_both = _sc_propagate(row2d, col2d, ew_p, xp)[0]

    zed = jnp.zeros((H_DIM, H_DIM), _f32)
    wc = jnp.concatenate([W_cz, W_cr, W_ch], axis=1)
    bc = jnp.concatenate([b_cz, b_cr, b_ch]).reshape(1, 3 * H_DIM)
    wbd = jnp.concatenate([
        jnp.concatenate([Wz[:H_DIM], zed, zed], axis=1),
        jnp.concatenate([zed, Wr[:H_DIM], zed], axis=1),
        jnp.concatenate([zed, zed, Wh[:H_DIM]], axis=1)], axis=0)
    whz = jnp.concatenate([Wz[H_DIM:], Wr[H_DIM:]], axis=1)
    bzr = jnp.concatenate([bz, br]).reshape(1, 2 * H_DIM)
    y, hn = _tc_dense(
        s_both[:N], s_both[NP:NP + N], dinv2d, xp, hidden_state,
        wc, bc, wbd, whz, bzr, Wh[H_DIM:], bh.reshape(1, H_DIM),
        W_out, b_out.reshape(1, F_IN))
    return (y, hn)


# speedup vs baseline: 1.0313x; 1.0001x over previous
import jax
import jax.numpy as jnp
from jax import lax
from jax.experimental import pallas as pl
from jax.experimental.pallas import tpu as pltpu
from jax.experimental.pallas import tpu_sc as plsc

N = 10000
E = 320000
F_IN = 128
H_DIM = 64

NC = 2
NS = 16
NW = NC * NS
NP = 10240
TS = NP // NS

E_PAD = 327680
E_ALL = E_PAD + 2048
EPW = E_PAD // NW
CH = 64
SUP = 1024
NSUP = EPW // SUP
CHA = 1024
NCHA = (E_PAD // NC // NS) // CHA

_f32 = jnp.float32
_i32 = jnp.int32


def _deg_body(col2d, ew_hbm, deg0_out, deg1_out,
              deg_sh, colA, ewA, zd):
    c = lax.axis_index("c")
    s = lax.axis_index("s")

    def zerod(i, carry):
        zd[pl.ds(i * 16, 16)] = jnp.zeros((16,), _f32)
        return carry
    lax.fori_loop(0, TS // 16, zerod, None)
    pltpu.sync_copy(zd, deg_sh.at[pl.ds(s * TS, TS)])
    plsc.subcore_barrier()

    def chunk(j, carry):
        base = (c * NS + s) * (E_PAD // NW) + j * CHA
        pltpu.sync_copy(
            col2d.at[pl.ds(pl.multiple_of(base // 128, 8), CHA // 128)], colA)
        pltpu.sync_copy(ew_hbm.at[pl.ds(pl.multiple_of(base, 8), CHA)], ewA)
        for jj in range(CHA // 128):
            pltpu.sync_copy(ewA.at[pl.ds(jj * 128, 128)],
                            deg_sh.at[colA.at[jj]], add=True)
        return carry
    lax.fori_loop(0, NCHA, chunk, None)
    plsc.subcore_barrier()

    @pl.when(c == 0)
    def _():
        pltpu.sync_copy(deg_sh.at[pl.ds(s * TS, TS)],
                        deg0_out.at[pl.ds(s * TS, TS)])

    @pl.when(c == 1)
    def _():
        pltpu.sync_copy(deg_sh.at[pl.ds(s * TS, TS)],
                        deg1_out.at[pl.ds(s * TS, TS)])


def _sc_degrees(col2d, ew_pad):
    kern = pl.kernel(
        _deg_body,
        out_type=[
            jax.ShapeDtypeStruct((NP,), _f32),
            jax.ShapeDtypeStruct((NP,), _f32),
        ],
        mesh=plsc.VectorSubcoreMesh(core_axis_name="c", subcore_axis_name="s"),
        compiler_params=pltpu.CompilerParams(needs_layout_passes=False),
        scratch_types=[
            pltpu.VMEM_SHARED((NP,), _f32),
            pltpu.VMEM((CHA // 128, 128), _i32),
            pltpu.VMEM((CHA,), _f32),
            pltpu.VMEM((TS,), _f32),
        ],
    )
    return kern(col2d, ew_pad)


def _prescale_body(deg0, deg1, x, xp_ref, dinv_ref):
    dv = lax.rsqrt(deg0[...] + deg1[...] + 1.0)
    dinv_ref[...] = dv
    xp_ref[...] = dv * x[...]


def _tc_prescale(deg0c, deg1c, x):
    BR = 2000
    row = lambda i: (i, 0)
    return pl.pallas_call(
        _prescale_body,
        grid=(N // BR,),
        in_specs=[
            pl.BlockSpec((BR, 1), row),
            pl.BlockSpec((BR, 1), row),
            pl.BlockSpec((BR, F_IN), row),
        ],
        out_specs=[
            pl.BlockSpec((BR, F_IN), row),
            pl.BlockSpec((BR, 1), row),
        ],
        out_shape=[
            jax.ShapeDtypeStruct((N, F_IN), _f32),
            jax.ShapeDtypeStruct((N, 1), _f32),
        ],
    )(deg0c, deg1c, x)


def _prop_body(row2d, col2d, ew_hbm, xp_hbm,
               s_out,
               s_sh, za,
               rowi0, rowi1, coli0, coli1, ewb0, ewb1,
               rows0, rows1, rows2, rows3,
               sg0, sg1, sg2, sg3, ss0, ss1, ss2, ss3, semi0, semi1):
    c = lax.axis_index("c")
    s = lax.axis_index("s")
    wid = s * NC + c
    rowi_b = (rowi0, rowi1)
    coli_b = (coli0, coli1)
    ew_b = (ewb0, ewb1)
    rows_b = (rows0, rows1, rows2, rows3)
    semg_b = (sg0, sg1, sg2, sg3)
    sems_b = (ss0, ss1, ss2, ss3)
    semi_b = (semi0, semi1)

    def zeroa(i, carry):
        for jj in range(F_IN // 16):
            za[i, pl.ds(jj * 16, 16)] = jnp.zeros((16,), _f32)
        return carry
    lax.fori_loop(0, 8, zeroa, None)
    for kk in range(TS // 8):
        pltpu.sync_copy(za, s_sh.at[pl.ds(s * TS + kk * 8, 8)])
    plsc.subcore_barrier()

    def stage_start(sc, b):
        base = pl.multiple_of(wid * (EPW // 128) + sc * (SUP // 128), 8)
        pltpu.async_copy(row2d.at[pl.ds(base, SUP // 128)], rowi_b[b],
                         semi_b[b])
        pltpu.async_copy(col2d.at[pl.ds(base, SUP // 128)], coli_b[b],
                         semi_b[b])
        ebase = pl.multiple_of(wid * EPW + sc * SUP, 8)
        pltpu.async_copy(ew_hbm.at[pl.ds(ebase, SUP)],
                         ew_b[b].at[pl.ds(0, SUP)], semi_b[b])

    def stage_wait(sc, b):
        base = pl.multiple_of(wid * (EPW // 128) + sc * (SUP // 128), 8)
        pltpu.make_async_copy(row2d.at[pl.ds(base, SUP // 128)], rowi_b[b],
                              semi_b[b]).wait()
        pltpu.make_async_copy(col2d.at[pl.ds(base, SUP // 128)], coli_b[b],
                              semi_b[b]).wait()
        ebase = pl.multiple_of(wid * EPW + sc * SUP, 8)
        pltpu.make_async_copy(ew_hbm.at[pl.ds(ebase, SUP)],
                              ew_b[b].at[pl.ds(0, SUP)], semi_b[b]).wait()

    def _idx(buf, sub):
        return buf.at[sub // 2, pl.ds((sub % 2) * CH, CH)]

    def gather_start(ib, sub, gb):
        pltpu.async_copy(xp_hbm.at[_idx(rowi_b[ib], sub)], rows_b[gb],
                         semg_b[gb])

    def gather_wait(ib, sub, gb):
        pltpu.make_async_copy(xp_hbm.at[_idx(rowi_b[ib], sub)], rows_b[gb],
                              semg_b[gb]).wait()

    def scatter_start(ib, sub, gb):
        pltpu.async_copy(rows_b[gb], s_sh.at[_idx(coli_b[ib], sub)],
                         sems_b[gb], add=True)

    def scatter_wait(ib, sub, gb):
        pltpu.make_async_copy(rows_b[gb], s_sh.at[_idx(coli_b[ib], sub)],
                              sems_b[gb]).wait()

    stage_start(jnp.int32(0), 0)
    stage_wait(jnp.int32(0), 0)
    gather_start(0, 0, 0)
    gather_start(0, 1, 1)
    stage_start(jnp.int32(1), 1)

    NSUB = SUP // CH

    def super_body(so, carry):
      for ib in range(2):
        sc = so * 2 + ib
        for sub in range(NSUB):
            gb = sub % 4
            gather_wait(ib, sub, gb)

            rows_ref = rows_b[gb]

            def mul_body(k0, inner):
                f16 = ew_b[ib][pl.ds(sub * CH + k0 * 8, 16)]
                for l in range(8):
                    k = k0 * 8 + l
                    fs = f16[l]
                    for jj in range(F_IN // 16):
                        rows_ref[k, pl.ds(jj * 16, 16)] = (
                            rows_ref[k, pl.ds(jj * 16, 16)] * fs)
                return inner
            lax.fori_loop(0, CH // 8, mul_body, None)

            ngb = (sub + 2) % 4
            if sub < 2:
                prev_ib = 1 - ib
                prev_sub = NSUB - 2 + sub

                @pl.when(sc > 0)
                def _():
                    scatter_wait(prev_ib, prev_sub, ngb)
            else:
                scatter_wait(ib, sub - 2, ngb)
            if sub < NSUB - 2:
                gather_start(ib, sub + 2, ngb)
            else:
                nsub = sub + 2 - NSUB
                if nsub == 0:
                    stage_wait(sc + 1, 1 - ib)
                gather_start(1 - ib, nsub, ngb)

            scatter_start(ib, sub, gb)
        stage_start(sc + 2, ib)
      return carry
    lax.fori_loop(0, NSUP // 2, super_body, None)

    ibf = NSUP % 2
    gather_wait(ibf, 0, 0)
    gather_wait(ibf, 1, 1)
    scatter_wait(1 - ibf, NSUB - 2, 2)
    scatter_wait(1 - ibf, NSUB - 1, 3)
    stage_wait(NSUP + 1, 1 - ibf)
    plsc.subcore_barrier()

    pltpu.sync_copy(
        s_sh.at[pl.ds(s * TS, TS)],
        s_out.at[pl.ds(pl.multiple_of(c * NP + s * TS, 8), TS)])


def _sc_propagate(row2d, col2d, ew_pad, xp):
    kern = pl.kernel(
        _prop_body,
        out_type=[
            jax.ShapeDtypeStruct((2 * NP, F_IN), _f32),
        ],
        mesh=plsc.VectorSubcoreMesh(core_axis_name="c", subcore_axis_name="s"),
        compiler_params=pltpu.CompilerParams(needs_layout_passes=False),
        scratch_types=[
            pltpu.VMEM_SHARED((NP, F_IN), _f32),
            pltpu.VMEM((8, F_IN), _f32),
            pltpu.VMEM((SUP // 128, 128), _i32),
            pltpu.VMEM((SUP // 128, 128), _i32),
            pltpu.VMEM((SUP // 128, 128), _i32),
            pltpu.VMEM((SUP // 128, 128), _i32),
            pltpu.VMEM((SUP + 16,), _f32),
            pltpu.VMEM((SUP + 16,), _f32),
            pltpu.VMEM((CH, F_IN), _f32),
            pltpu.VMEM((CH, F_IN), _f32),
            pltpu.VMEM((CH, F_IN), _f32),
            pltpu.VMEM((CH, F_IN), _f32),
            pltpu.SemaphoreType.DMA,
            pltpu.SemaphoreType.DMA,
            pltpu.SemaphoreType.DMA,
            pltpu.SemaphoreType.DMA,
            pltpu.SemaphoreType.DMA,
            pltpu.SemaphoreType.DMA,
            pltpu.SemaphoreType.DMA,
            pltpu.SemaphoreType.DMA,
            pltpu.SemaphoreType.DMA,
            pltpu.SemaphoreType.DMA,
        ],
    )
    return kern(row2d, col2d, ew_pad, xp)


def _tc_body(s0, s1, dinv, xp, h,
             wc, bc, wbd, whz, bzr, wh_b, bh,
             w_out, b_out, y_ref, hn_ref):
    dv = dinv[...]
    xa = dv * (s0[...] + s1[...] + xp[...])
    hh = h[...]
    c = jnp.dot(xa, wc[...]) + bc[...]
    g = jnp.dot(c, wbd[...])
    t = jnp.dot(hh, whz[...]) + bzr[...]
    z = jax.nn.sigmoid(g[:, :H_DIM] + t[:, :H_DIM])
    r = jax.nn.sigmoid(g[:, H_DIM:2 * H_DIM] + t[:, H_DIM:])
    ht = jnp.tanh(g[:, 2 * H_DIM:] + jnp.dot(hh * r, wh_b[...]) + bh[...])
    hn = z * hh + (1.0 - z) * ht
    hn_ref[...] = hn
    y_ref[...] = jnp.dot(jax.nn.relu(hn), w_out[...]) + b_out[...]


def _tc_dense(s0, s1, dinv2d, xp, h, wc, bc, wbd, whz, bzr, wh_b, bh,
              w_out, b_out):
    BR = 2000
    row = lambda i: (i, 0)
    rep = lambda i: (0, 0)
    return pl.pallas_call(
        _tc_body,
        grid=(N // BR,),
        in_specs=[
            pl.BlockSpec((BR, F_IN), row),
            pl.BlockSpec((BR, F_IN), row),
            pl.BlockSpec((BR, 1), row),
            pl.BlockSpec((BR, F_IN), row),
            pl.BlockSpec((BR, H_DIM), row),
            pl.BlockSpec((F_IN, 3 * H_DIM), rep),
            pl.BlockSpec((1, 3 * H_DIM), rep),
            pl.BlockSpec((3 * H_DIM, 3 * H_DIM), rep),
            pl.BlockSpec((H_DIM, 2 * H_DIM), rep),
            pl.BlockSpec((1, 2 * H_DIM), rep),
            pl.BlockSpec((H_DIM, H_DIM), rep),
            pl.BlockSpec((1, H_DIM), rep),
            pl.BlockSpec((H_DIM, F_IN), rep),
            pl.BlockSpec((1, F_IN), rep),
        ],
        out_specs=[
            pl.BlockSpec((BR, F_IN), row),
            pl.BlockSpec((BR, H_DIM), row),
        ],
        out_shape=[
            jax.ShapeDtypeStruct((N, F_IN), _f32),
            jax.ShapeDtypeStruct((N, H_DIM), _f32),
        ],
    )(s0, s1, dinv2d, xp, h, wc, bc, wbd, whz, bzr, wh_b, bh, w_out, b_out)


def kernel(g_edge_index, node_feat, edge_weight, hidden_state,
           W_cz, b_cz, Wz, bz, W_cr, b_cr, Wr, br,
           W_ch, b_ch, Wh, bh, W_out, b_out):
    row = g_edge_index[0]
    col = g_edge_index[1]

    npad = E_ALL - E
    pad_idx = (lax.iota(_i32, npad) * 37) % N
    row_p = jnp.concatenate([row, pad_idx])
    col_p = jnp.concatenate([col, pad_idx])
    ew_p = jnp.concatenate([edge_weight, jnp.zeros((npad,), _f32)])
    row2d = row_p.reshape(E_ALL // 128, 128)
    col2d = col_p.reshape(E_ALL // 128, 128)

    deg0, deg1 = _sc_degrees(col2d, ew_p)
    xp, dinv2d = _tc_prescale(deg0[:N].reshape(N, 1),
                              deg1[:N].reshape(N, 1), node_feat)
    s_both = _sc_propagate(row2d, col2d, ew_p, xp)[0]

    zed = jnp.zeros((H_DIM, H_DIM), _f32)
    wc = jnp.concatenate([W_cz, W_cr, W_ch], axis=1)
    bc = jnp.concatenate([b_cz, b_cr, b_ch]).reshape(1, 3 * H_DIM)
    wbd = jnp.concatenate([
        jnp.concatenate([Wz[:H_DIM], zed, zed], axis=1),
        jnp.concatenate([zed, Wr[:H_DIM], zed], axis=1),
        jnp.concatenate([zed, zed, Wh[:H_DIM]], axis=1)], axis=0)
    whz = jnp.concatenate([Wz[H_DIM:], Wr[H_DIM:]], axis=1)
    bzr = jnp.concatenate([bz, br]).reshape(1, 2 * H_DIM)
    y, hn = _tc_dense(
        s_both[:N], s_both[NP:NP + N], dinv2d, xp, hidden_state,
        wc, bc, wbd, whz, bzr, Wh[H_DIM:], bh.reshape(1, H_DIM),
        W_out, b_out.reshape(1, F_IN))
    return (y, hn)


# speedup vs baseline: 1.9056x; 1.8479x over previous
import jax
import jax.numpy as jnp
from jax import lax
from jax.experimental import pallas as pl
from jax.experimental.pallas import tpu as pltpu
from jax.experimental.pallas import tpu_sc as plsc

N = 10000
E = 320000
F_IN = 128
H_DIM = 64

NC = 2
NS = 16
NW = NC * NS
NP = 10240
TS = NP // NS

E_PAD = 327680
E_ALL = E_PAD + 2048
EPW = E_PAD // NW
CH = 128
SUP = 1024
NSUP = EPW // SUP
CHA = 1024
NCHA = (E_PAD // NC // NS) // CHA

_f32 = jnp.float32
_i32 = jnp.int32


def _deg_body(col2d, ew_hbm, deg0_out, deg1_out,
              deg_sh, colA, ewA, zd):
    c = lax.axis_index("c")
    s = lax.axis_index("s")

    def zerod(i, carry):
        zd[pl.ds(i * 16, 16)] = jnp.zeros((16,), _f32)
        return carry
    lax.fori_loop(0, TS // 16, zerod, None)
    pltpu.sync_copy(zd, deg_sh.at[pl.ds(s * TS, TS)])
    plsc.subcore_barrier()

    def chunk(j, carry):
        base = (c * NS + s) * (E_PAD // NW) + j * CHA
        pltpu.sync_copy(
            col2d.at[pl.ds(pl.multiple_of(base // 128, 8), CHA // 128)], colA)
        pltpu.sync_copy(ew_hbm.at[pl.ds(pl.multiple_of(base, 8), CHA)], ewA)
        for jj in range(CHA // 128):
            pltpu.sync_copy(ewA.at[pl.ds(jj * 128, 128)],
                            deg_sh.at[colA.at[jj]], add=True)
        return carry
    lax.fori_loop(0, NCHA, chunk, None)
    plsc.subcore_barrier()

    @pl.when(c == 0)
    def _():
        pltpu.sync_copy(deg_sh.at[pl.ds(s * TS, TS)],
                        deg0_out.at[pl.ds(s * TS, TS)])

    @pl.when(c == 1)
    def _():
        pltpu.sync_copy(deg_sh.at[pl.ds(s * TS, TS)],
                        deg1_out.at[pl.ds(s * TS, TS)])


def _sc_degrees(col2d, ew_pad):
    kern = pl.kernel(
        _deg_body,
        out_type=[
            jax.ShapeDtypeStruct((NP,), _f32),
            jax.ShapeDtypeStruct((NP,), _f32),
        ],
        mesh=plsc.VectorSubcoreMesh(core_axis_name="c", subcore_axis_name="s"),
        compiler_params=pltpu.CompilerParams(needs_layout_passes=False),
        scratch_types=[
            pltpu.VMEM_SHARED((NP,), _f32),
            pltpu.VMEM((CHA // 128, 128), _i32),
            pltpu.VMEM((CHA,), _f32),
            pltpu.VMEM((TS,), _f32),
        ],
    )
    return kern(col2d, ew_pad)


def _prescale_body(deg0, deg1, x, xp_ref, dinv_ref):
    dv = lax.rsqrt(deg0[...] + deg1[...] + 1.0)
    dinv_ref[...] = dv
    xp_ref[...] = dv * x[...]


def _tc_prescale(deg0c, deg1c, x):
    BR = 2000
    row = lambda i: (i, 0)
    return pl.pallas_call(
        _prescale_body,
        grid=(N // BR,),
        in_specs=[
            pl.BlockSpec((BR, 1), row),
            pl.BlockSpec((BR, 1), row),
            pl.BlockSpec((BR, F_IN), row),
        ],
        out_specs=[
            pl.BlockSpec((BR, F_IN), row),
            pl.BlockSpec((BR, 1), row),
        ],
        out_shape=[
            jax.ShapeDtypeStruct((N, F_IN), _f32),
            jax.ShapeDtypeStruct((N, 1), _f32),
        ],
    )(deg0c, deg1c, x)


def _prop_body(row2d, col2d, ew_hbm, xp_hbm,
               s_out,
               s_sh, za,
               rowi0, rowi1, coli0, coli1, ewb0, ewb1,
               rows0, rows1, sg0, sg1, semi0, semi1):
    c = lax.axis_index("c")
    s = lax.axis_index("s")
    wid = s * NC + c
    rowi_b = (rowi0, rowi1)
    coli_b = (coli0, coli1)
    ew_b = (ewb0, ewb1)
    rows_b = (rows0, rows1)
    semg_b = (sg0, sg1)
    semi_b = (semi0, semi1)

    def zeroa(i, carry):
        for jj in range(F_IN // 16):
            za[i, pl.ds(jj * 16, 16)] = jnp.zeros((16,), _f32)
        return carry
    lax.fori_loop(0, 8, zeroa, None)
    for kk in range(TS // 8):
        pltpu.sync_copy(za, s_sh.at[pl.ds(s * TS + kk * 8, 8)])
    plsc.subcore_barrier()

    def stage_start(sc, b):
        base = pl.multiple_of(wid * (EPW // 128) + sc * (SUP // 128), 8)
        pltpu.async_copy(row2d.at[pl.ds(base, SUP // 128)], rowi_b[b],
                         semi_b[b])
        pltpu.async_copy(col2d.at[pl.ds(base, SUP // 128)], coli_b[b],
                         semi_b[b])
        ebase = pl.multiple_of(wid * EPW + sc * SUP, 8)
        pltpu.async_copy(ew_hbm.at[pl.ds(ebase, SUP)],
                         ew_b[b].at[pl.ds(0, SUP)], semi_b[b])

    def stage_wait(sc, b):
        base = pl.multiple_of(wid * (EPW // 128) + sc * (SUP // 128), 8)
        pltpu.make_async_copy(row2d.at[pl.ds(base, SUP // 128)], rowi_b[b],
                              semi_b[b]).wait()
        pltpu.make_async_copy(col2d.at[pl.ds(base, SUP // 128)], coli_b[b],
                              semi_b[b]).wait()
        ebase = pl.multiple_of(wid * EPW + sc * SUP, 8)
        pltpu.make_async_copy(ew_hbm.at[pl.ds(ebase, SUP)],
                              ew_b[b].at[pl.ds(0, SUP)], semi_b[b]).wait()

    def gather_start(ib, sub, gb):
        pltpu.async_copy(xp_hbm.at[rowi_b[ib].at[sub]], rows_b[gb],
                         semg_b[gb])

    def gather_wait(ib, sub, gb):
        pltpu.make_async_copy(xp_hbm.at[rowi_b[ib].at[sub]], rows_b[gb],
                              semg_b[gb]).wait()

    stage_start(jnp.int32(0), 0)
    stage_wait(jnp.int32(0), 0)
    gather_start(0, 0, 0)
    gather_start(0, 1, 1)
    stage_start(jnp.int32(1), 1)

    NSUB = SUP // CH

    def super_body(so, carry):
      for ib in range(2):
        sc = so * 2 + ib
        for sub in range(NSUB):
            gb = sub % 2
            gather_wait(ib, sub, gb)

            rows_ref = rows_b[gb]

            def mul_body(k0, inner):
                f16 = ew_b[ib][pl.ds(sub * CH + k0 * 16, 16)]
                for l in range(16):
                    k = k0 * 16 + l
                    fs = f16[l]
                    for jj in range(F_IN // 16):
                        rows_ref[k, pl.ds(jj * 16, 16)] = (
                            rows_ref[k, pl.ds(jj * 16, 16)] * fs)
                return inner
            lax.fori_loop(0, CH // 16, mul_body, None)

            pltpu.sync_copy(rows_ref, s_sh.at[coli_b[ib].at[sub]], add=True)

            if sub < NSUB - 2:
                gather_start(ib, sub + 2, gb)
            else:
                nsub = sub + 2 - NSUB
                if nsub == 0:
                    stage_wait(sc + 1, 1 - ib)
                gather_start(1 - ib, nsub, gb)
        stage_start(sc + 2, ib)
      return carry
    lax.fori_loop(0, NSUP // 2, super_body, None)

    ibf = NSUP % 2
    gather_wait(ibf, 0, 0)
    gather_wait(ibf, 1, 1)
    stage_wait(NSUP + 1, 1 - ibf)
    plsc.subcore_barrier()

    pltpu.sync_copy(
        s_sh.at[pl.ds(s * TS, TS)],
        s_out.at[pl.ds(pl.multiple_of(c * NP + s * TS, 8), TS)])


def _sc_propagate(row2d, col2d, ew_pad, xp):
    kern = pl.kernel(
        _prop_body,
        out_type=[
            jax.ShapeDtypeStruct((2 * NP, F_IN), _f32),
        ],
        mesh=plsc.VectorSubcoreMesh(core_axis_name="c", subcore_axis_name="s"),
        compiler_params=pltpu.CompilerParams(needs_layout_passes=False),
        scratch_types=[
            pltpu.VMEM_SHARED((NP, F_IN), _f32),
            pltpu.VMEM((8, F_IN), _f32),
            pltpu.VMEM((SUP // 128, 128), _i32),
            pltpu.VMEM((SUP // 128, 128), _i32),
            pltpu.VMEM((SUP // 128, 128), _i32),
            pltpu.VMEM((SUP // 128, 128), _i32),
            pltpu.VMEM((SUP + 16,), _f32),
            pltpu.VMEM((SUP + 16,), _f32),
            pltpu.VMEM((CH, F_IN), _f32),
            pltpu.VMEM((CH, F_IN), _f32),
            pltpu.SemaphoreType.DMA,
            pltpu.SemaphoreType.DMA,
            pltpu.SemaphoreType.DMA,
            pltpu.SemaphoreType.DMA,
        ],
    )
    return kern(row2d, col2d, ew_pad, xp)


def _tc_body(s0, s1, dinv, xp, h,
             wc, bc, wbd, whz, bzr, wh_b, bh,
             w_out, b_out, y_ref, hn_ref):
    dv = dinv[...]
    xa = dv * (s0[...] + s1[...] + xp[...])
    hh = h[...]
    c = jnp.dot(xa, wc[...]) + bc[...]
    g = jnp.dot(c, wbd[...])
    t = jnp.dot(hh, whz[...]) + bzr[...]
    z = jax.nn.sigmoid(g[:, :H_DIM] + t[:, :H_DIM])
    r = jax.nn.sigmoid(g[:, H_DIM:2 * H_DIM] + t[:, H_DIM:])
    ht = jnp.tanh(g[:, 2 * H_DIM:] + jnp.dot(hh * r, wh_b[...]) + bh[...])
    hn = z * hh + (1.0 - z) * ht
    hn_ref[...] = hn
    y_ref[...] = jnp.dot(jax.nn.relu(hn), w_out[...]) + b_out[...]


def _tc_dense(s0, s1, dinv2d, xp, h, wc, bc, wbd, whz, bzr, wh_b, bh,
              w_out, b_out):
    BR = 2000
    row = lambda i: (i, 0)
    rep = lambda i: (0, 0)
    return pl.pallas_call(
        _tc_body,
        grid=(N // BR,),
        in_specs=[
            pl.BlockSpec((BR, F_IN), row),
            pl.BlockSpec((BR, F_IN), row),
            pl.BlockSpec((BR, 1), row),
            pl.BlockSpec((BR, F_IN), row),
            pl.BlockSpec((BR, H_DIM), row),
            pl.BlockSpec((F_IN, 3 * H_DIM), rep),
            pl.BlockSpec((1, 3 * H_DIM), rep),
            pl.BlockSpec((3 * H_DIM, 3 * H_DIM), rep),
            pl.BlockSpec((H_DIM, 2 * H_DIM), rep),
            pl.BlockSpec((1, 2 * H_DIM), rep),
            pl.BlockSpec((H_DIM, H_DIM), rep),
            pl.BlockSpec((1, H_DIM), rep),
            pl.BlockSpec((H_DIM, F_IN), rep),
            pl.BlockSpec((1, F_IN), rep),
        ],
        out_specs=[
            pl.BlockSpec((BR, F_IN), row),
            pl.BlockSpec((BR, H_DIM), row),
        ],
        out_shape=[
            jax.ShapeDtypeStruct((N, F_IN), _f32),
            jax.ShapeDtypeStruct((N, H_DIM), _f32),
        ],
    )(s0, s1, dinv2d, xp, h, wc, bc, wbd, whz, bzr, wh_b, bh, w_out, b_out)


def kernel(g_edge_index, node_feat, edge_weight, hidden_state,
           W_cz, b_cz, Wz, bz, W_cr, b_cr, Wr, br,
           W_ch, b_ch, Wh, bh, W_out, b_out):
    row = g_edge_index[0]
    col = g_edge_index[1]

    npad = E_ALL - E
    pad_idx = (lax.iota(_i32, npad) * 37) % N
    row_p = jnp.concatenate([row, pad_idx])
    col_p = jnp.concatenate([col, pad_idx])
    ew_p = jnp.concatenate([edge_weight, jnp.zeros((npad,), _f32)])
    row2d = row_p.reshape(E_ALL // 128, 128)
    col2d = col_p.reshape(E_ALL // 128, 128)

    deg0, deg1 = _sc_degrees(col2d, ew_p)
    xp, dinv2d = _tc_prescale(deg0[:N].reshape(N, 1),
                              deg1[:N].reshape(N, 1), node_feat)
    s_both = _sc_propagate(row2d, col2d, ew_p, xp)[0]

    zed = jnp.zeros((H_DIM, H_DIM), _f32)
    wc = jnp.concatenate([W_cz, W_cr, W_ch], axis=1)
    bc = jnp.concatenate([b_cz, b_cr, b_ch]).reshape(1, 3 * H_DIM)
    wbd = jnp.concatenate([
        jnp.concatenate([Wz[:H_DIM], zed, zed], axis=1),
        jnp.concatenate([zed, Wr[:H_DIM], zed], axis=1),
        jnp.concatenate([zed, zed, Wh[:H_DIM]], axis=1)], axis=0)
    whz = jnp.concatenate([Wz[H_DIM:], Wr[H_DIM:]], axis=1)
    bzr = jnp.concatenate([bz, br]).reshape(1, 2 * H_DIM)
    y, hn = _tc_dense(
        s_both[:N], s_both[NP:NP + N], dinv2d, xp, hidden_state,
        wc, bc, wbd, whz, bzr, Wh[H_DIM:], bh.reshape(1, H_DIM),
        W_out, b_out.reshape(1, F_IN))
    return (y, hn)


# speedup vs baseline: 1.9134x; 1.0041x over previous
import jax
import jax.numpy as jnp
from jax import lax
from jax.experimental import pallas as pl
from jax.experimental.pallas import tpu as pltpu
from jax.experimental.pallas import tpu_sc as plsc

N = 10000
E = 320000
F_IN = 128
H_DIM = 64

NC = 2
NS = 16
NW = NC * NS
NP = 10240
TS = NP // NS

E_PAD = 327680
E_ALL = E_PAD + 2048
EPW = E_PAD // NW
CH = 128
SUP = 1024
NSUP = EPW // SUP
CHA = 1024
NCHA = (E_PAD // NC // NS) // CHA

_f32 = jnp.float32
_i32 = jnp.int32


def _deg_body(col2d, ew_hbm, deg0_out, deg1_out,
              deg_sh, colA, ewA, zd):
    c = lax.axis_index("c")
    s = lax.axis_index("s")

    def zerod(i, carry):
        zd[pl.ds(i * 16, 16)] = jnp.zeros((16,), _f32)
        return carry
    lax.fori_loop(0, TS // 16, zerod, None)
    pltpu.sync_copy(zd, deg_sh.at[pl.ds(s * TS, TS)])
    plsc.subcore_barrier()

    def chunk(j, carry):
        base = (c * NS + s) * (E_PAD // NW) + j * CHA
        pltpu.sync_copy(
            col2d.at[pl.ds(pl.multiple_of(base // 128, 8), CHA // 128)], colA)
        pltpu.sync_copy(ew_hbm.at[pl.ds(pl.multiple_of(base, 8), CHA)], ewA)
        for jj in range(CHA // 128):
            pltpu.sync_copy(ewA.at[pl.ds(jj * 128, 128)],
                            deg_sh.at[colA.at[jj]], add=True)
        return carry
    lax.fori_loop(0, NCHA, chunk, None)
    plsc.subcore_barrier()

    @pl.when(c == 0)
    def _():
        pltpu.sync_copy(deg_sh.at[pl.ds(s * TS, TS)],
                        deg0_out.at[pl.ds(s * TS, TS)])

    @pl.when(c == 1)
    def _():
        pltpu.sync_copy(deg_sh.at[pl.ds(s * TS, TS)],
                        deg1_out.at[pl.ds(s * TS, TS)])


def _sc_degrees(col2d, ew_pad):
    kern = pl.kernel(
        _deg_body,
        out_type=[
            jax.ShapeDtypeStruct((NP,), _f32),
            jax.ShapeDtypeStruct((NP,), _f32),
        ],
        mesh=plsc.VectorSubcoreMesh(core_axis_name="c", subcore_axis_name="s"),
        compiler_params=pltpu.CompilerParams(needs_layout_passes=False),
        scratch_types=[
            pltpu.VMEM_SHARED((NP,), _f32),
            pltpu.VMEM((CHA // 128, 128), _i32),
            pltpu.VMEM((CHA,), _f32),
            pltpu.VMEM((TS,), _f32),
        ],
    )
    return kern(col2d, ew_pad)


def _prescale_body(deg0, deg1, x, xp_ref, dinv_ref):
    dv = lax.rsqrt(deg0[...] + deg1[...] + 1.0)
    dinv_ref[...] = dv
    xp_ref[...] = dv * x[...]


def _tc_prescale(deg0c, deg1c, x):
    BR = 2000
    row = lambda i: (i, 0)
    return pl.pallas_call(
        _prescale_body,
        grid=(N // BR,),
        in_specs=[
            pl.BlockSpec((BR, 1), row),
            pl.BlockSpec((BR, 1), row),
            pl.BlockSpec((BR, F_IN), row),
        ],
        out_specs=[
            pl.BlockSpec((BR, F_IN), row),
            pl.BlockSpec((BR, 1), row),
        ],
        out_shape=[
            jax.ShapeDtypeStruct((N, F_IN), _f32),
            jax.ShapeDtypeStruct((N, 1), _f32),
        ],
    )(deg0c, deg1c, x)


def _prop_body(row2d, col2d, ew_hbm, xp_hbm,
               s_out,
               s_sh, za,
               rowi0, rowi1, coli0, coli1, ewb0, ewb1,
               rows0, rows1, sg0, sg1, semi0, semi1):
    c = lax.axis_index("c")
    s = lax.axis_index("s")
    wid = s * NC + c
    rowi_b = (rowi0, rowi1)
    coli_b = (coli0, coli1)
    ew_b = (ewb0, ewb1)
    rows_b = (rows0, rows1)
    semg_b = (sg0, sg1)
    semi_b = (semi0, semi1)

    def zeroa(i, carry):
        for jj in range(F_IN // 16):
            za[i, pl.ds(jj * 16, 16)] = jnp.zeros((16,), _f32)
        return carry
    lax.fori_loop(0, 8, zeroa, None)
    for kk in range(TS // 8):
        pltpu.sync_copy(za, s_sh.at[pl.ds(s * TS + kk * 8, 8)])
    plsc.subcore_barrier()

    def stage_start(sc, b):
        base = pl.multiple_of(wid * (EPW // 128) + sc * (SUP // 128), 8)
        pltpu.async_copy(row2d.at[pl.ds(base, SUP // 128)], rowi_b[b],
                         semi_b[b])
        pltpu.async_copy(col2d.at[pl.ds(base, SUP // 128)], coli_b[b],
                         semi_b[b])
        ebase = pl.multiple_of(wid * EPW + sc * SUP, 8)
        pltpu.async_copy(ew_hbm.at[pl.ds(ebase, SUP)],
                         ew_b[b].at[pl.ds(0, SUP)], semi_b[b])

    def stage_wait(sc, b):
        base = pl.multiple_of(wid * (EPW // 128) + sc * (SUP // 128), 8)
        pltpu.make_async_copy(row2d.at[pl.ds(base, SUP // 128)], rowi_b[b],
                              semi_b[b]).wait()
        pltpu.make_async_copy(col2d.at[pl.ds(base, SUP // 128)], coli_b[b],
                              semi_b[b]).wait()
        ebase = pl.multiple_of(wid * EPW + sc * SUP, 8)
        pltpu.make_async_copy(ew_hbm.at[pl.ds(ebase, SUP)],
                              ew_b[b].at[pl.ds(0, SUP)], semi_b[b]).wait()

    def gather_start(ib, sub, gb):
        pltpu.async_copy(xp_hbm.at[rowi_b[ib].at[sub]], rows_b[gb],
                         semg_b[gb])

    def gather_wait(ib, sub, gb):
        pltpu.make_async_copy(xp_hbm.at[rowi_b[ib].at[sub]], rows_b[gb],
                              semg_b[gb]).wait()

    stage_start(jnp.int32(0), 0)
    stage_wait(jnp.int32(0), 0)
    gather_start(0, 0, 0)
    gather_start(0, 1, 1)
    stage_start(jnp.int32(1), 1)

    NSUB = SUP // CH

    def super_body(so, carry):
      for ib in range(2):
        sc = so * 2 + ib
        for sub in range(NSUB):
            gb = sub % 2
            gather_wait(ib, sub, gb)

            rows_ref = rows_b[gb]

            @plsc.parallel_loop(0, CH // 16, step=1, unroll=2)
            def _mul(k0):
                f16 = ew_b[ib][pl.ds(sub * CH + k0 * 16, 16)]
                for l in range(16):
                    k = k0 * 16 + l
                    fs = f16[l]
                    for jj in range(F_IN // 16):
                        rows_ref[k, pl.ds(jj * 16, 16)] = (
                            rows_ref[k, pl.ds(jj * 16, 16)] * fs)

            pltpu.sync_copy(rows_ref, s_sh.at[coli_b[ib].at[sub]], add=True)

            if sub < NSUB - 2:
                gather_start(ib, sub + 2, gb)
            else:
                nsub = sub + 2 - NSUB
                if nsub == 0:
                    stage_wait(sc + 1, 1 - ib)
                gather_start(1 - ib, nsub, gb)
        stage_start(sc + 2, ib)
      return carry
    lax.fori_loop(0, NSUP // 2, super_body, None)

    ibf = NSUP % 2
    gather_wait(ibf, 0, 0)
    gather_wait(ibf, 1, 1)
    stage_wait(NSUP + 1, 1 - ibf)
    plsc.subcore_barrier()

    pltpu.sync_copy(
        s_sh.at[pl.ds(s * TS, TS)],
        s_out.at[pl.ds(pl.multiple_of(c * NP + s * TS, 8), TS)])


def _sc_propagate(row2d, col2d, ew_pad, xp):
    kern = pl.kernel(
        _prop_body,
        out_type=[
            jax.ShapeDtypeStruct((2 * NP, F_IN), _f32),
        ],
        mesh=plsc.VectorSubcoreMesh(core_axis_name="c", subcore_axis_name="s"),
        compiler_params=pltpu.CompilerParams(needs_layout_passes=False),
        scratch_types=[
            pltpu.VMEM_SHARED((NP, F_IN), _f32),
            pltpu.VMEM((8, F_IN), _f32),
            pltpu.VMEM((SUP // 128, 128), _i32),
            pltpu.VMEM((SUP // 128, 128), _i32),
            pltpu.VMEM((SUP // 128, 128), _i32),
            pltpu.VMEM((SUP // 128, 128), _i32),
            pltpu.VMEM((SUP + 16,), _f32),
            pltpu.VMEM((SUP + 16,), _f32),
            pltpu.VMEM((CH, F_IN), _f32),
            pltpu.VMEM((CH, F_IN), _f32),
            pltpu.SemaphoreType.DMA,
            pltpu.SemaphoreType.DMA,
            pltpu.SemaphoreType.DMA,
            pltpu.SemaphoreType.DMA,
        ],
    )
    return kern(row2d, col2d, ew_pad, xp)


def _tc_body(s0, s1, dinv, xp, h,
             wc, bc, wbd, whz, bzr, wh_b, bh,
             w_out, b_out, y_ref, hn_ref):
    dv = dinv[...]
    xa = dv * (s0[...] + s1[...] + xp[...])
    hh = h[...]
    c = jnp.dot(xa, wc[...]) + bc[...]
    g = jnp.dot(c, wbd[...])
    t = jnp.dot(hh, whz[...]) + bzr[...]
    z = jax.nn.sigmoid(g[:, :H_DIM] + t[:, :H_DIM])
    r = jax.nn.sigmoid(g[:, H_DIM:2 * H_DIM] + t[:, H_DIM:])
    ht = jnp.tanh(g[:, 2 * H_DIM:] + jnp.dot(hh * r, wh_b[...]) + bh[...])
    hn = z * hh + (1.0 - z) * ht
    hn_ref[...] = hn
    y_ref[...] = jnp.dot(jax.nn.relu(hn), w_out[...]) + b_out[...]


def _tc_dense(s0, s1, dinv2d, xp, h, wc, bc, wbd, whz, bzr, wh_b, bh,
              w_out, b_out):
    BR = 2000
    row = lambda i: (i, 0)
    rep = lambda i: (0, 0)
    return pl.pallas_call(
        _tc_body,
        grid=(N // BR,),
        in_specs=[
            pl.BlockSpec((BR, F_IN), row),
            pl.BlockSpec((BR, F_IN), row),
            pl.BlockSpec((BR, 1), row),
            pl.BlockSpec((BR, F_IN), row),
            pl.BlockSpec((BR, H_DIM), row),
            pl.BlockSpec((F_IN, 3 * H_DIM), rep),
            pl.BlockSpec((1, 3 * H_DIM), rep),
            pl.BlockSpec((3 * H_DIM, 3 * H_DIM), rep),
            pl.BlockSpec((H_DIM, 2 * H_DIM), rep),
            pl.BlockSpec((1, 2 * H_DIM), rep),
            pl.BlockSpec((H_DIM, H_DIM), rep),
            pl.BlockSpec((1, H_DIM), rep),
            pl.BlockSpec((H_DIM, F_IN), rep),
            pl.BlockSpec((1, F_IN), rep),
        ],
        out_specs=[
            pl.BlockSpec((BR, F_IN), row),
            pl.BlockSpec((BR, H_DIM), row),
        ],
        out_shape=[
            jax.ShapeDtypeStruct((N, F_IN), _f32),
            jax.ShapeDtypeStruct((N, H_DIM), _f32),
        ],
    )(s0, s1, dinv2d, xp, h, wc, bc, wbd, whz, bzr, wh_b, bh, w_out, b_out)


def kernel(g_edge_index, node_feat, edge_weight, hidden_state,
           W_cz, b_cz, Wz, bz, W_cr, b_cr, Wr, br,
           W_ch, b_ch, Wh, bh, W_out, b_out):
    row = g_edge_index[0]
    col = g_edge_index[1]

    npad = E_ALL - E
    pad_idx = (lax.iota(_i32, npad) * 37) % N
    row_p = jnp.concatenate([row, pad_idx])
    col_p = jnp.concatenate([col, pad_idx])
    ew_p = jnp.concatenate([edge_weight, jnp.zeros((npad,), _f32)])
    row2d = row_p.reshape(E_ALL // 128, 128)
    col2d = col_p.reshape(E_ALL // 128, 128)

    deg0, deg1 = _sc_degrees(col2d, ew_p)
    xp, dinv2d = _tc_prescale(deg0[:N].reshape(N, 1),
                              deg1[:N].reshape(N, 1), node_feat)
    s_both = _sc_propagate(row2d, col2d, ew_p, xp)[0]

    zed = jnp.zeros((H_DIM, H_DIM), _f32)
    wc = jnp.concatenate([W_cz, W_cr, W_ch], axis=1)
    bc = jnp.concatenate([b_cz, b_cr, b_ch]).reshape(1, 3 * H_DIM)
    wbd = jnp.concatenate([
        jnp.concatenate([Wz[:H_DIM], zed, zed], axis=1),
        jnp.concatenate([zed, Wr[:H_DIM], zed], axis=1),
        jnp.concatenate([zed, zed, Wh[:H_DIM]], axis=1)], axis=0)
    whz = jnp.concatenate([Wz[H_DIM:], Wr[H_DIM:]], axis=1)
    bzr = jnp.concatenate([bz, br]).reshape(1, 2 * H_DIM)
    y, hn = _tc_dense(
        s_both[:N], s_both[NP:NP + N], dinv2d, xp, hidden_state,
        wc, bc, wbd, whz, bzr, Wh[H_DIM:], bh.reshape(1, H_DIM),
        W_out, b_out.reshape(1, F_IN))
    return (y, hn)


# speedup vs baseline: 1.9344x; 1.0110x over previous
import jax
import jax.numpy as jnp
from jax import lax
from jax.experimental import pallas as pl
from jax.experimental.pallas import tpu as pltpu
from jax.experimental.pallas import tpu_sc as plsc

N = 10000
E = 320000
F_IN = 128
H_DIM = 64

NC = 2
NS = 16
NW = NC * NS
NP = 10240
TS = NP // NS

E_PAD = 327680
E_ALL = E_PAD + 2048
EPW = E_PAD // NW
CH = 128
SUP = 1024
NSUP = EPW // SUP
CHA = 2048
NCHA = (E_PAD // NC // NS) // CHA

_f32 = jnp.float32
_i32 = jnp.int32


def _deg_body(col2d, ew_hbm, deg0_out, deg1_out,
              deg_sh, colA, ewA, zd):
    c = lax.axis_index("c")
    s = lax.axis_index("s")

    def zerod(i, carry):
        zd[pl.ds(i * 16, 16)] = jnp.zeros((16,), _f32)
        return carry
    lax.fori_loop(0, TS // 16, zerod, None)
    pltpu.sync_copy(zd, deg_sh.at[pl.ds(s * TS, TS)])
    plsc.subcore_barrier()

    def chunk(j, carry):
        base = (c * NS + s) * (E_PAD // NW) + j * CHA
        pltpu.sync_copy(
            col2d.at[pl.ds(pl.multiple_of(base // 128, 8), CHA // 128)], colA)
        pltpu.sync_copy(ew_hbm.at[pl.ds(pl.multiple_of(base, 8), CHA)], ewA)
        for jj in range(CHA // 128):
            pltpu.sync_copy(ewA.at[pl.ds(jj * 128, 128)],
                            deg_sh.at[colA.at[jj]], add=True)
        return carry
    lax.fori_loop(0, NCHA, chunk, None)
    plsc.subcore_barrier()

    @pl.when(c == 0)
    def _():
        pltpu.sync_copy(deg_sh.at[pl.ds(s * TS, TS)],
                        deg0_out.at[pl.ds(s * TS, TS)])

    @pl.when(c == 1)
    def _():
        pltpu.sync_copy(deg_sh.at[pl.ds(s * TS, TS)],
                        deg1_out.at[pl.ds(s * TS, TS)])


def _sc_degrees(col2d, ew_pad):
    kern = pl.kernel(
        _deg_body,
        out_type=[
            jax.ShapeDtypeStruct((NP,), _f32),
            jax.ShapeDtypeStruct((NP,), _f32),
        ],
        mesh=plsc.VectorSubcoreMesh(core_axis_name="c", subcore_axis_name="s"),
        compiler_params=pltpu.CompilerParams(needs_layout_passes=False),
        scratch_types=[
            pltpu.VMEM_SHARED((NP,), _f32),
            pltpu.VMEM((CHA // 128, 128), _i32),
            pltpu.VMEM((CHA,), _f32),
            pltpu.VMEM((TS,), _f32),
        ],
    )
    return kern(col2d, ew_pad)


def _prescale_body(deg0, deg1, x, xp_ref, dinv_ref):
    dv = lax.rsqrt(deg0[...] + deg1[...] + 1.0)
    dinv_ref[...] = dv
    xp_ref[...] = dv * x[...]


def _tc_prescale(deg0c, deg1c, x):
    BR = 2000
    row = lambda i: (i, 0)
    return pl.pallas_call(
        _prescale_body,
        grid=(N // BR,),
        in_specs=[
            pl.BlockSpec((BR, 1), row),
            pl.BlockSpec((BR, 1), row),
            pl.BlockSpec((BR, F_IN), row),
        ],
        out_specs=[
            pl.BlockSpec((BR, F_IN), row),
            pl.BlockSpec((BR, 1), row),
        ],
        out_shape=[
            jax.ShapeDtypeStruct((N, F_IN), _f32),
            jax.ShapeDtypeStruct((N, 1), _f32),
        ],
    )(deg0c, deg1c, x)


def _prop_body(row2d, col2d, ew_hbm, xp_hbm,
               s_out,
               s_sh, za,
               rowi0, rowi1, coli0, coli1, ewb0, ewb1,
               rows0, rows1, sg0, sg1, semi0, semi1):
    c = lax.axis_index("c")
    s = lax.axis_index("s")
    wid = s * NC + c
    rowi_b = (rowi0, rowi1)
    coli_b = (coli0, coli1)
    ew_b = (ewb0, ewb1)
    rows_b = (rows0, rows1)
    semg_b = (sg0, sg1)
    semi_b = (semi0, semi1)

    def zeroa(i, carry):
        for jj in range(F_IN // 16):
            za[i, pl.ds(jj * 16, 16)] = jnp.zeros((16,), _f32)
        return carry
    lax.fori_loop(0, 8, zeroa, None)
    for kk in range(TS // 8):
        pltpu.sync_copy(za, s_sh.at[pl.ds(s * TS + kk * 8, 8)])
    plsc.subcore_barrier()

    def stage_start(sc, b):
        base = pl.multiple_of(wid * (EPW // 128) + sc * (SUP // 128), 8)
        pltpu.async_copy(row2d.at[pl.ds(base, SUP // 128)], rowi_b[b],
                         semi_b[b])
        pltpu.async_copy(col2d.at[pl.ds(base, SUP // 128)], coli_b[b],
                         semi_b[b])
        ebase = pl.multiple_of(wid * EPW + sc * SUP, 8)
        pltpu.async_copy(ew_hbm.at[pl.ds(ebase, SUP)],
                         ew_b[b].at[pl.ds(0, SUP)], semi_b[b])

    def stage_wait(sc, b):
        base = pl.multiple_of(wid * (EPW // 128) + sc * (SUP // 128), 8)
        pltpu.make_async_copy(row2d.at[pl.ds(base, SUP // 128)], rowi_b[b],
                              semi_b[b]).wait()
        pltpu.make_async_copy(col2d.at[pl.ds(base, SUP // 128)], coli_b[b],
                              semi_b[b]).wait()
        ebase = pl.multiple_of(wid * EPW + sc * SUP, 8)
        pltpu.make_async_copy(ew_hbm.at[pl.ds(ebase, SUP)],
                              ew_b[b].at[pl.ds(0, SUP)], semi_b[b]).wait()

    def gather_start(ib, sub, gb):
        pltpu.async_copy(xp_hbm.at[rowi_b[ib].at[sub]], rows_b[gb],
                         semg_b[gb])

    def gather_wait(ib, sub, gb):
        pltpu.make_async_copy(xp_hbm.at[rowi_b[ib].at[sub]], rows_b[gb],
                              semg_b[gb]).wait()

    stage_start(jnp.int32(0), 0)
    stage_wait(jnp.int32(0), 0)
    gather_start(0, 0, 0)
    gather_start(0, 1, 1)
    stage_start(jnp.int32(1), 1)

    NSUB = SUP // CH

    def super_body(so, carry):
      for ib in range(2):
        sc = so * 2 + ib
        for sub in range(NSUB):
            gb = sub % 2
            gather_wait(ib, sub, gb)

            rows_ref = rows_b[gb]

            @plsc.parallel_loop(0, CH // 16, step=1, unroll=2)
            def _mul(k0):
                f16 = ew_b[ib][pl.ds(sub * CH + k0 * 16, 16)]
                for l in range(16):
                    k = k0 * 16 + l
                    fs = f16[l]
                    for jj in range(F_IN // 16):
                        rows_ref[k, pl.ds(jj * 16, 16)] = (
                            rows_ref[k, pl.ds(jj * 16, 16)] * fs)

            pltpu.sync_copy(rows_ref, s_sh.at[coli_b[ib].at[sub]], add=True)

            if sub < NSUB - 2:
                gather_start(ib, sub + 2, gb)
            else:
                nsub = sub + 2 - NSUB
                if nsub == 0:
                    stage_wait(sc + 1, 1 - ib)
                gather_start(1 - ib, nsub, gb)
        stage_start(sc + 2, ib)
      return carry
    lax.fori_loop(0, NSUP // 2, super_body, None)

    ibf = NSUP % 2
    gather_wait(ibf, 0, 0)
    gather_wait(ibf, 1, 1)
    stage_wait(NSUP + 1, 1 - ibf)
    plsc.subcore_barrier()

    pltpu.sync_copy(
        s_sh.at[pl.ds(s * TS, TS)],
        s_out.at[pl.ds(pl.multiple_of(c * NP + s * TS, 8), TS)])


def _sc_propagate(row2d, col2d, ew_pad, xp):
    kern = pl.kernel(
        _prop_body,
        out_type=[
            jax.ShapeDtypeStruct((2 * NP, F_IN), _f32),
        ],
        mesh=plsc.VectorSubcoreMesh(core_axis_name="c", subcore_axis_name="s"),
        compiler_params=pltpu.CompilerParams(needs_layout_passes=False),
        scratch_types=[
            pltpu.VMEM_SHARED((NP, F_IN), _f32),
            pltpu.VMEM((8, F_IN), _f32),
            pltpu.VMEM((SUP // 128, 128), _i32),
            pltpu.VMEM((SUP // 128, 128), _i32),
            pltpu.VMEM((SUP // 128, 128), _i32),
            pltpu.VMEM((SUP // 128, 128), _i32),
            pltpu.VMEM((SUP + 16,), _f32),
            pltpu.VMEM((SUP + 16,), _f32),
            pltpu.VMEM((CH, F_IN), _f32),
            pltpu.VMEM((CH, F_IN), _f32),
            pltpu.SemaphoreType.DMA,
            pltpu.SemaphoreType.DMA,
            pltpu.SemaphoreType.DMA,
            pltpu.SemaphoreType.DMA,
        ],
    )
    return kern(row2d, col2d, ew_pad, xp)


def _tc_body(s0, s1, dinv, xp, h,
             wc, bc, wbd, whz, bzr, wh_b, bh,
             w_out, b_out, y_ref, hn_ref):
    dv = dinv[...]
    xa = dv * (s0[...] + s1[...] + xp[...])
    hh = h[...]
    c = jnp.dot(xa, wc[...]) + bc[...]
    g = jnp.dot(c, wbd[...])
    t = jnp.dot(hh, whz[...]) + bzr[...]
    z = jax.nn.sigmoid(g[:, :H_DIM] + t[:, :H_DIM])
    r = jax.nn.sigmoid(g[:, H_DIM:2 * H_DIM] + t[:, H_DIM:])
    ht = jnp.tanh(g[:, 2 * H_DIM:] + jnp.dot(hh * r, wh_b[...]) + bh[...])
    hn = z * hh + (1.0 - z) * ht
    hn_ref[...] = hn
    y_ref[...] = jnp.dot(jax.nn.relu(hn), w_out[...]) + b_out[...]


def _tc_dense(s0, s1, dinv2d, xp, h, wc, bc, wbd, whz, bzr, wh_b, bh,
              w_out, b_out):
    BR = 1000
    row = lambda i: (i, 0)
    rep = lambda i: (0, 0)
    return pl.pallas_call(
        _tc_body,
        grid=(N // BR,),
        in_specs=[
            pl.BlockSpec((BR, F_IN), row),
            pl.BlockSpec((BR, F_IN), row),
            pl.BlockSpec((BR, 1), row),
            pl.BlockSpec((BR, F_IN), row),
            pl.BlockSpec((BR, H_DIM), row),
            pl.BlockSpec((F_IN, 3 * H_DIM), rep),
            pl.BlockSpec((1, 3 * H_DIM), rep),
            pl.BlockSpec((3 * H_DIM, 3 * H_DIM), rep),
            pl.BlockSpec((H_DIM, 2 * H_DIM), rep),
            pl.BlockSpec((1, 2 * H_DIM), rep),
            pl.BlockSpec((H_DIM, H_DIM), rep),
            pl.BlockSpec((1, H_DIM), rep),
            pl.BlockSpec((H_DIM, F_IN), rep),
            pl.BlockSpec((1, F_IN), rep),
        ],
        out_specs=[
            pl.BlockSpec((BR, F_IN), row),
            pl.BlockSpec((BR, H_DIM), row),
        ],
        out_shape=[
            jax.ShapeDtypeStruct((N, F_IN), _f32),
            jax.ShapeDtypeStruct((N, H_DIM), _f32),
        ],
    )(s0, s1, dinv2d, xp, h, wc, bc, wbd, whz, bzr, wh_b, bh, w_out, b_out)


def kernel(g_edge_index, node_feat, edge_weight, hidden_state,
           W_cz, b_cz, Wz, bz, W_cr, b_cr, Wr, br,
           W_ch, b_ch, Wh, bh, W_out, b_out):
    row = g_edge_index[0]
    col = g_edge_index[1]

    npad = E_ALL - E
    pad_idx = (lax.iota(_i32, npad) * 37) % N
    row_p = jnp.concatenate([row, pad_idx])
    col_p = jnp.concatenate([col, pad_idx])
    ew_p = jnp.concatenate([edge_weight, jnp.zeros((npad,), _f32)])
    row2d = row_p.reshape(E_ALL // 128, 128)
    col2d = col_p.reshape(E_ALL // 128, 128)

    deg0, deg1 = _sc_degrees(col2d, ew_p)
    xp, dinv2d = _tc_prescale(deg0[:N].reshape(N, 1),
                              deg1[:N].reshape(N, 1), node_feat)
    s_both = _sc_propagate(row2d, col2d, ew_p, xp)[0]

    zed = jnp.zeros((H_DIM, H_DIM), _f32)
    wc = jnp.concatenate([W_cz, W_cr, W_ch], axis=1)
    bc = jnp.concatenate([b_cz, b_cr, b_ch]).reshape(1, 3 * H_DIM)
    wbd = jnp.concatenate([
        jnp.concatenate([Wz[:H_DIM], zed, zed], axis=1),
        jnp.concatenate([zed, Wr[:H_DIM], zed], axis=1),
        jnp.concatenate([zed, zed, Wh[:H_DIM]], axis=1)], axis=0)
    whz = jnp.concatenate([Wz[H_DIM:], Wr[H_DIM:]], axis=1)
    bzr = jnp.concatenate([bz, br]).reshape(1, 2 * H_DIM)
    y, hn = _tc_dense(
        s_both[:N], s_both[NP:NP + N], dinv2d, xp, hidden_state,
        wc, bc, wbd, whz, bzr, Wh[H_DIM:], bh.reshape(1, H_DIM),
        W_out, b_out.reshape(1, F_IN))
    return (y, hn)
